# baseline (speedup 1.0000x reference)
import functools

import jax
import jax.numpy as jnp
from jax import lax
from jax.experimental import pallas as pl
from jax.experimental.pallas import tpu as pltpu
from jax.experimental.pallas import tpu_sc as plsc

LANES = 16
TB = 8192


@functools.cache
def _build_detile_kernel(V, D):
    grid = -(-V // TB)

    HW = D // 2

    def body(x_ref, o_ref):
        x = x_ref[...].astype(jnp.bfloat16)
        ci = lax.broadcasted_iota(jnp.int32, (D, HW * 4), 0)
        ki = lax.broadcasted_iota(jnp.int32, (D, HW * 4), 1)
        e_lo = [((ci < HW) & (ki == ci + HW * u)).astype(jnp.bfloat16)
                for u in range(4)]
        e_hi = [((ci >= HW) & (ki == ci - HW + HW * u)).astype(jnp.bfloat16)
                for u in range(4)]
        dn = (((0,), (0,)), ((), ()))
        for p in range(TB // 1024):
            halves = []
            for s in (2 * p, 2 * p + 1):
                lo = jnp.zeros((128, HW * 4), jnp.float32)
                hi = jnp.zeros((128, HW * 4), jnp.float32)
                for u in range(4):
                    xu = x[:, s * 512 + u * 128:s * 512 + (u + 1) * 128]
                    lo = lo + lax.dot_general(
                        xu, e_lo[u], dn, preferred_element_type=jnp.float32)
                    hi = hi + lax.dot_general(
                        xu, e_hi[u], dn, preferred_element_type=jnp.float32)
                lob = lax.bitcast_convert_type(lo, jnp.uint32)
                hib = lax.bitcast_convert_type(hi, jnp.uint32)
                w = (lob >> 16) | (hib & jnp.uint32(0xFFFF0000))
                halves.append(w)
            o_ref[pl.ds(p * 128, 128), :] = jnp.concatenate(halves, axis=1)

    rows = TB // 8
    return pl.pallas_call(
        body,
        grid=(grid,),
        in_specs=[pl.BlockSpec((D, TB), lambda i: (0, i))],
        out_specs=pl.BlockSpec((rows, 128), lambda i: (i, 0)),
        out_shape=jax.ShapeDtypeStruct((grid * rows, 128), jnp.uint32),
    )


@functools.cache
def _build_pool_kernel(B, L, D, VP, CH):
    info = plsc.get_sparse_core_info()
    NC, NS = info.num_cores, info.num_subcores
    NW = NC * NS
    RPW = B // NW
    NCH = RPW // CH
    CI = CH * L
    SEGS = [(0, CI)]
    W = D // 2
    inv_l = 1.0 / L

    mesh = plsc.VectorSubcoreMesh(core_axis_name="c", subcore_axis_name="s")

    @functools.partial(
        pl.kernel,
        mesh=mesh,
        out_type=jax.ShapeDtypeStruct((B, D), jnp.float32),
        compiler_params=pltpu.CompilerParams(
            use_tc_tiling_on_sc=False, needs_layout_passes=False),
        scratch_types=[
            pltpu.VMEM((CI,), jnp.int32),
            pltpu.VMEM((CI,), jnp.int32),
            pltpu.VMEM((CH * L, W), jnp.uint32),
            pltpu.VMEM((CH * L, W), jnp.uint32),
            pltpu.VMEM((RPW, D), jnp.float32),
            pltpu.SemaphoreType.DMA,
            pltpu.SemaphoreType.DMA,
        ],
    )
    def body(ids_hbm, table_hbm, out_hbm,
             idx_a, idx_b, rows_a, rows_b, out_v, sem_a, sem_b):
        wid = lax.axis_index("s") * NC + lax.axis_index("c")
        wbase = wid * RPW

        def copy_idx(c, idxv):
            start = (wbase + c * CH) * L
            pltpu.sync_copy(ids_hbm.at[pl.ds(start, CI)], idxv)

        def fire(idxv, rowsv, sem):
            for o, n in SEGS:
                pltpu.async_copy(table_hbm.at[idxv.at[pl.ds(o, n)]],
                                 rowsv.at[pl.ds(o, n)], sem)

        def drain(idxv, rowsv, sem):
            for o, n in SEGS:
                pltpu.make_async_copy(table_hbm.at[idxv.at[pl.ds(o, n)]],
                                      rowsv.at[pl.ds(o, n)], sem).wait()

        def accum(c, rowsv):
            zero = jnp.zeros((LANES,), jnp.float32)
            hmask = jnp.uint32(0xFFFF0000)
            for o in range(CH):
                def inner(r, carry, _o=o):
                    lo = list(carry[:4])
                    hi = list(carry[4:])
                    r0 = _o * L + 8 * r
                    for k in range(4):
                        xa = rowsv[r0 + 2 * k, :]
                        xb = rowsv[r0 + 2 * k + 1, :]
                        lo[k] = lo[k] + plsc.bitcast(xa << 16, jnp.float32)
                        hi[k] = hi[k] + plsc.bitcast(xa & hmask, jnp.float32)
                        lo[k] = lo[k] + plsc.bitcast(xb << 16, jnp.float32)
                        hi[k] = hi[k] + plsc.bitcast(xb & hmask, jnp.float32)
                    return tuple(lo) + tuple(hi)
                acc = lax.fori_loop(0, L // 8, inner, (zero,) * 8)
                row = c * CH + o
                out_v[row, pl.ds(0, LANES)] = (
                    (acc[0] + acc[1]) + (acc[2] + acc[3])) * inv_l
                out_v[row, pl.ds(LANES, LANES)] = (
                    (acc[4] + acc[5]) + (acc[6] + acc[7])) * inv_l

        copy_idx(0, idx_a)
        fire(idx_a, rows_a, sem_a)

        def step2(g, carry):
            c0 = 2 * g
            copy_idx(c0 + 1, idx_b)
            fire(idx_b, rows_b, sem_b)
            drain(idx_a, rows_a, sem_a)
            accum(c0, rows_a)

            @pl.when(c0 + 2 < NCH)
            def _():
                copy_idx(c0 + 2, idx_a)
                fire(idx_a, rows_a, sem_a)

            drain(idx_b, rows_b, sem_b)
            accum(c0 + 1, rows_b)
            return carry

        lax.fori_loop(0, NCH // 2, step2, 0)
        pltpu.sync_copy(out_v, out_hbm.at[pl.ds(wbase, RPW)])

    return body


def kernel(token_ids, token_emb_weight, null_context):
    B, L = token_ids.shape
    V, D = token_emb_weight.shape
    detile = _build_detile_kernel(V, D)
    lin = detile(token_emb_weight.T)
    VP = lin.size * 2 // D
    table_w = lin.reshape(VP, D // 2)
    t = token_ids.astype(jnp.int32)
    g = (t & ~jnp.int32(1023)) | ((t & 127) << 3) | ((t >> 7) & 7)
    ids = g.reshape(B * L)
    pool = _build_pool_kernel(B, L, D, VP, CH=8)
    return pool(ids, table_w)


# speedup vs baseline: 1.0419x; 1.0419x over previous
import functools

import jax
import jax.numpy as jnp
from jax import lax
from jax.experimental import pallas as pl
from jax.experimental.pallas import tpu as pltpu
from jax.experimental.pallas import tpu_sc as plsc

LANES = 16
TB = 8192


@functools.cache
def _build_detile_kernel(V, D):
    grid = -(-V // TB)

    HW = D // 2

    def body(x_ref, o_ref):
        x = x_ref[...].astype(jnp.bfloat16)
        ci = lax.broadcasted_iota(jnp.int32, (D, HW * 4), 0)
        ki = lax.broadcasted_iota(jnp.int32, (D, HW * 4), 1)
        e_lo = [((ci < HW) & (ki == ci + HW * u)).astype(jnp.bfloat16)
                for u in range(4)]
        e_hi = [((ci >= HW) & (ki == ci - HW + HW * u)).astype(jnp.bfloat16)
                for u in range(4)]
        dn = (((0,), (0,)), ((), ()))
        for p in range(TB // 1024):
            halves = []
            for s in (2 * p, 2 * p + 1):
                lo = jnp.zeros((128, HW * 4), jnp.float32)
                hi = jnp.zeros((128, HW * 4), jnp.float32)
                for u in range(4):
                    xu = x[:, s * 512 + u * 128:s * 512 + (u + 1) * 128]
                    lo = lo + lax.dot_general(
                        xu, e_lo[u], dn, preferred_element_type=jnp.float32)
                    hi = hi + lax.dot_general(
                        xu, e_hi[u], dn, preferred_element_type=jnp.float32)
                lob = lax.bitcast_convert_type(lo, jnp.uint32)
                hib = lax.bitcast_convert_type(hi, jnp.uint32)
                w = (lob >> 16) | (hib & jnp.uint32(0xFFFF0000))
                halves.append(w)
            o_ref[pl.ds(p * 128, 128), :] = jnp.concatenate(halves, axis=1)

    rows = TB // 8
    return pl.pallas_call(
        body,
        grid=(grid,),
        in_specs=[pl.BlockSpec((D, TB), lambda i: (0, i))],
        out_specs=pl.BlockSpec((rows, 128), lambda i: (i, 0)),
        out_shape=jax.ShapeDtypeStruct((grid * rows, 128), jnp.uint32),
    )


@functools.cache
def _build_pool_kernel(B, L, D, VP, CH):
    info = plsc.get_sparse_core_info()
    NC, NS = info.num_cores, info.num_subcores
    NW = NC * NS
    RPW = B // NW
    NCH = RPW // CH
    CI = CH * L
    SEGS = [(0, CI)]
    W = D // 2
    inv_l = 1.0 / L

    mesh = plsc.VectorSubcoreMesh(core_axis_name="c", subcore_axis_name="s")

    @functools.partial(
        pl.kernel,
        mesh=mesh,
        out_type=jax.ShapeDtypeStruct((B, D), jnp.float32),
        compiler_params=pltpu.CompilerParams(
            use_tc_tiling_on_sc=False, needs_layout_passes=False),
        scratch_types=[
            pltpu.VMEM((CI,), jnp.int32),
            pltpu.VMEM((CI,), jnp.int32),
            pltpu.VMEM((CH * L, W), jnp.uint32),
            pltpu.VMEM((CH * L, W), jnp.uint32),
            pltpu.VMEM((RPW, D), jnp.float32),
            pltpu.SemaphoreType.DMA,
            pltpu.SemaphoreType.DMA,
        ],
    )
    def body(ids_hbm, table_hbm, out_hbm,
             idx_a, idx_b, rows_a, rows_b, out_v, sem_a, sem_b):
        wid = lax.axis_index("s") * NC + lax.axis_index("c")
        wbase = wid * RPW

        def copy_idx(c, idxv):
            start = (wbase + c * CH) * L
            pltpu.sync_copy(ids_hbm.at[pl.ds(start, CI)], idxv)

        def fire(idxv, rowsv, sem):
            for o, n in SEGS:
                pltpu.async_copy(table_hbm.at[idxv.at[pl.ds(o, n)]],
                                 rowsv.at[pl.ds(o, n)], sem)

        def drain(idxv, rowsv, sem):
            for o, n in SEGS:
                pltpu.make_async_copy(table_hbm.at[idxv.at[pl.ds(o, n)]],
                                      rowsv.at[pl.ds(o, n)], sem).wait()

        def accum(c, rowsv):
            zero = jnp.zeros((LANES,), jnp.float32)
            hmask = jnp.uint32(0xFFFF0000)
            for o in range(CH):
                def inner(r, carry, _o=o):
                    lo = list(carry[:4])
                    hi = list(carry[4:])
                    r0 = _o * L + 8 * r
                    for k in range(4):
                        xa = rowsv[r0 + 2 * k, :]
                        xb = rowsv[r0 + 2 * k + 1, :]
                        lo[k] = lo[k] + plsc.bitcast(xa << 16, jnp.float32)
                        hi[k] = hi[k] + plsc.bitcast(xa & hmask, jnp.float32)
                        lo[k] = lo[k] + plsc.bitcast(xb << 16, jnp.float32)
                        hi[k] = hi[k] + plsc.bitcast(xb & hmask, jnp.float32)
                    return tuple(lo) + tuple(hi)
                acc = lax.fori_loop(0, L // 8, inner, (zero,) * 8)
                row = c * CH + o
                out_v[row, pl.ds(0, LANES)] = (
                    (acc[0] + acc[1]) + (acc[2] + acc[3])) * inv_l
                out_v[row, pl.ds(LANES, LANES)] = (
                    (acc[4] + acc[5]) + (acc[6] + acc[7])) * inv_l

        copy_idx(0, idx_a)
        fire(idx_a, rows_a, sem_a)

        def step2(g, carry):
            c0 = 2 * g
            copy_idx(c0 + 1, idx_b)
            fire(idx_b, rows_b, sem_b)
            drain(idx_a, rows_a, sem_a)
            accum(c0, rows_a)

            @pl.when(c0 + 2 < NCH)
            def _():
                copy_idx(c0 + 2, idx_a)
                fire(idx_a, rows_a, sem_a)

            drain(idx_b, rows_b, sem_b)
            accum(c0 + 1, rows_b)
            return carry

        lax.fori_loop(0, NCH // 2, step2, 0)
        pltpu.sync_copy(out_v, out_hbm.at[pl.ds(wbase, RPW)])

    return body


def kernel(token_ids, token_emb_weight, null_context):
    B, L = token_ids.shape
    V, D = token_emb_weight.shape
    detile = _build_detile_kernel(V, D)
    lin = detile(token_emb_weight.T)
    VP = lin.size * 2 // D
    table_w = lin.reshape(VP, D // 2)
    t = token_ids.astype(jnp.int32)
    g = (t & ~jnp.int32(1023)) | ((t & 127) << 3) | ((t >> 7) & 7)
    ids = g.reshape(B * L)
    pool = _build_pool_kernel(B, L, D, VP, CH=16)
    return pool(ids, table_w)
